# baseline (speedup 1.0000x reference)
import functools

import jax
import jax.numpy as jnp
from jax import lax
from jax.experimental import pallas as pl
from jax.experimental.pallas import tpu as pltpu
from jax.experimental.pallas import tpu_sc as plsc

B, H, W_IN, C = 1, 256, 256, 96
KS = 2
N = B * H * W_IN * C
OUT = (H * KS) * (W_IN * KS) * C
NC, NS, L = 2, 16, 16
NWIN = 32
WIN = OUT // NWIN
PASSES = NWIN // NC
CHUNK = 8192
SHARE = N // NS
NCHUNK = SHARE // CHUNK
NPAIR = NCHUNK // 2
WSLICE = WIN // NS


def _unpool_sc(packed_ext):
    mesh = plsc.VectorSubcoreMesh(core_axis_name="c", subcore_axis_name="s")

    @functools.partial(
        pl.kernel,
        mesh=mesh,
        out_type=jax.ShapeDtypeStruct((OUT,), jnp.float32),
        scratch_types=[
            pltpu.VMEM((2 * CHUNK,), jnp.float32),
            pltpu.VMEM((2 * CHUNK,), jnp.float32),
            pltpu.VMEM((CHUNK,), jnp.int32),
            pltpu.VMEM((CHUNK,), jnp.int32),
            pltpu.VMEM((CHUNK,), jnp.float32),
            pltpu.VMEM((CHUNK,), jnp.float32),
            pltpu.VMEM_SHARED((WIN,), jnp.float32),
            pltpu.SemaphoreType.DMA,
            pltpu.SemaphoreType.DMA,
            pltpu.SemaphoreType.DMA,
            pltpu.SemaphoreType.DMA,
        ],
    )
    def k(pk_hbm, out_hbm,
          pb0, pb1, si0, si1, sv0, sv1, win_sh,
          lsem0, lsem1, ssem0, ssem1):
        c = lax.axis_index("c")
        s = lax.axis_index("s")
        lanes = lax.iota(jnp.int32, L)
        pb = (pb0, pb1)
        sib = (si0, si1)
        svb = (sv0, sv1)
        lsem = (lsem0, lsem1)
        ssem = (ssem0, ssem1)

        def start_load(kk, b):
            base = (s * NCHUNK + kk) * (2 * CHUNK)
            pltpu.async_copy(pk_hbm.at[pl.ds(base, 2 * CHUNK)], pb[b], lsem[b])

        def wait_load(kk, b):
            base = (s * NCHUNK + kk) * (2 * CHUNK)
            pltpu.make_async_copy(pk_hbm.at[pl.ds(base, 2 * CHUNK)],
                                  pb[b], lsem[b]).wait()

        def start_scat(b):
            pltpu.async_copy(svb[b], win_sh.at[sib[b]], ssem[b], add=True)

        def wait_scat(b):
            pltpu.make_async_copy(svb[b], win_sh.at[sib[b]], ssem[b]).wait()

        def do_pass(p, carry):
            lo = (p * NC + c) * WIN

            pltpu.sync_copy(pk_hbm.at[pl.ds(2 * N, WSLICE)],
                            win_sh.at[pl.ds(s * WSLICE, WSLICE)])
            plsc.subcore_barrier()

            start_load(0, 0)
            start_load(1, 1)

            def do_chunk(kk2, cy):
                for b in (0, 1):
                    kk = kk2 * 2 + b
                    wait_load(kk, b)

                    @pl.when(kk >= 2)
                    def _():
                        wait_scat(b)

                    ib = pb[b]
                    si = sib[b]
                    sv = svb[b]

                    def vec(i, cz):
                        o = i * L
                        rel = jax.lax.bitcast_convert_type(ib[pl.ds(o, L)], jnp.int32) - lo
                        vv = ib[pl.ds(CHUNK + o, L)]
                        ok = (rel >= 0) & (rel < WIN)
                        pad = (s * CHUNK + o) + lanes
                        si[pl.ds(o, L)] = jnp.where(ok, rel, pad)
                        sv[pl.ds(o, L)] = jnp.where(ok, vv, 0.0)
                        return cz
                    lax.fori_loop(0, CHUNK // L, vec, None)

                    @pl.when(kk + 2 < NCHUNK)
                    def _():
                        start_load(kk + 2, b)
                    start_scat(b)
                return cy
            lax.fori_loop(0, NPAIR, do_chunk, None)
            wait_scat(0)
            wait_scat(1)
            plsc.subcore_barrier()

            o = s * WSLICE
            pltpu.sync_copy(win_sh.at[pl.ds(o, WSLICE)],
                            out_hbm.at[pl.ds(lo + o, WSLICE)])
            plsc.subcore_barrier()
            return carry
        lax.fori_loop(0, PASSES, do_pass, None)

    return k(packed_ext)


def kernel(pool, ind, k_size):
    pool_flat = pool.reshape(N)
    ind_flat = ind.reshape(N) + (jnp.asarray(k_size, jnp.int32) - KS)
    ind_f = jax.lax.bitcast_convert_type(ind_flat, jnp.float32)
    packed = jnp.stack([ind_f.reshape(-1, CHUNK),
                        pool_flat.reshape(-1, CHUNK)], axis=1).reshape(-1)
    packed_ext = jnp.concatenate([packed, jnp.zeros((WSLICE,), jnp.float32)])
    out = _unpool_sc(packed_ext)
    return out.reshape(B, H * KS, W_IN * KS, C)
